# Initial kernel scaffold; baseline (speedup 1.0000x reference)
import functools

import jax
import jax.numpy as jnp
from jax import lax
from jax.experimental import pallas as pl
from jax.experimental.pallas import tpu as pltpu
from jax.experimental.pallas import tpu_sc as plsc

N_NODES = 10000
N_EDGES = 320000
N_GRAPHS = 64
HIDDEN = 128
EPS = 1e-5

_NC = 2
_NS = 16
_NW = _NC * _NS
_K = 80
_NB = N_EDGES // (_NW * _K)
_RPT = N_NODES // _NS
_ZROWS = 125

_mesh = plsc.VectorSubcoreMesh(
    core_axis_name="c", subcore_axis_name="s", num_cores=_NC, num_subcores=_NS
)


def _sc_deg(row3):
  @functools.partial(
      pl.kernel,
      out_type=jax.ShapeDtypeStruct((_NC, N_NODES, 16), jnp.float32),
      mesh=_mesh,
      scratch_types=[
          pltpu.VMEM((_NB, _K), jnp.int32),
          pltpu.VMEM((_K, 16), jnp.float32),
          pltpu.VMEM((_RPT, 16), jnp.float32),
          pltpu.VMEM_SHARED((N_NODES, 16), jnp.float32),
          pltpu.SemaphoreType.DMA,
      ],
  )
  def k(row_hbm, out_hbm, idx_v, ones_v, zslab_v, acc, sem):
    cid = lax.axis_index("c")
    sid = lax.axis_index("s")
    wid = cid * _NS + sid

    def fill_ones(i, _):
      ones_v[i, :] = jnp.full((16,), 1.0, jnp.float32)
      return 0

    lax.fori_loop(0, _K, fill_ones, 0)

    def fill_z(i, _):
      zslab_v[i, :] = jnp.zeros((16,), jnp.float32)
      return 0

    lax.fori_loop(0, _RPT, fill_z, 0)
    pltpu.sync_copy(zslab_v, acc.at[pl.ds(sid * _RPT, _RPT)])
    pltpu.sync_copy(row_hbm.at[wid], idx_v)
    plsc.subcore_barrier()

    def blk(i, _):
      pltpu.sync_copy(ones_v, acc.at[idx_v.at[i]], add=True)
      return 0

    lax.fori_loop(0, _NB, blk, 0)
    plsc.subcore_barrier()
    pltpu.sync_copy(
        acc.at[pl.ds(sid * _RPT, _RPT)],
        out_hbm.at[cid, pl.ds(sid * _RPT, _RPT)],
    )

  return k(row3)


def _sc_agg(u, row3, col3):
  @functools.partial(
      pl.kernel,
      out_type=jax.ShapeDtypeStruct((_NC, N_NODES, HIDDEN), jnp.float32),
      mesh=_mesh,
      scratch_types=[
          pltpu.VMEM((_NB, _K), jnp.int32),
          pltpu.VMEM((_NB, _K), jnp.int32),
          pltpu.VMEM((_K, HIDDEN), jnp.float32),
          pltpu.VMEM((_ZROWS, HIDDEN), jnp.float32),
          pltpu.VMEM_SHARED((N_NODES, HIDDEN), jnp.float32),
          pltpu.SemaphoreType.DMA,
      ],
  )
  def k(u_hbm, row_hbm, col_hbm, out_hbm, rowv, colv, gbuf, zslab_v, acc, sem):
    cid = lax.axis_index("c")
    sid = lax.axis_index("s")
    wid = cid * _NS + sid

    def fill_z(i, _):
      for c in range(HIDDEN // 16):
        zslab_v[i, pl.ds(c * 16, 16)] = jnp.zeros((16,), jnp.float32)
      return 0

    lax.fori_loop(0, _ZROWS, fill_z, 0)
    for j in range(_RPT // _ZROWS):
      pltpu.sync_copy(zslab_v, acc.at[pl.ds(sid * _RPT + j * _ZROWS, _ZROWS)])
    pltpu.sync_copy(row_hbm.at[wid], rowv)
    pltpu.sync_copy(col_hbm.at[wid], colv)
    plsc.subcore_barrier()

    def blk(i, _):
      pltpu.async_copy(u_hbm.at[colv.at[i]], gbuf, sem).wait()
      pltpu.sync_copy(gbuf, acc.at[rowv.at[i]], add=True)
      return 0

    lax.fori_loop(0, _NB, blk, 0)
    plsc.subcore_barrier()
    pltpu.sync_copy(
        acc.at[pl.ds(sid * _RPT, _RPT)],
        out_hbm.at[cid, pl.ds(sid * _RPT, _RPT)],
    )

  return k(u, row3, col3)


def _bn_in(x, g, b):
  m = jnp.mean(x, axis=0)
  xc = x - m
  v = jnp.mean(xc * xc, axis=0)
  return g * xc * lax.rsqrt(v + EPS) + b


def _dinv_in(degp_ref):
  deg = degp_ref[0, :, 0:1] + degp_ref[1, :, 0:1] + 1.0
  return lax.rsqrt(deg)


def _tc_stage_a(x, bfg, bfb, Wf, bf, g1, b1b, W1, degp):
  def body(x_ref, bfg_ref, bfb_ref, Wf_ref, bf_ref, g1_ref, b1b_ref, W1_ref,
           degp_ref, u_ref):
    h = _bn_in(x_ref[...], bfg_ref[...], bfb_ref[...])
    h = jnp.maximum(
        jnp.dot(h, Wf_ref[...], preferred_element_type=jnp.float32)
        + bf_ref[...], 0.0)
    hb = _bn_in(h, g1_ref[...], b1b_ref[...])
    t = jnp.dot(hb, W1_ref[...], preferred_element_type=jnp.float32)
    u_ref[...] = _dinv_in(degp_ref) * t

  return pl.pallas_call(
      body, out_shape=jax.ShapeDtypeStruct((N_NODES, HIDDEN), jnp.float32)
  )(x, bfg, bfb, Wf, bf, g1, b1b, W1, degp)


def _tc_stage_b(p, u, degp, b, g, be, W):
  def body(p_ref, u_ref, degp_ref, b_ref, g_ref, be_ref, W_ref, o_ref):
    dinv = _dinv_in(degp_ref)
    agg = dinv * (p_ref[0] + p_ref[1] + u_ref[...])
    h = jnp.maximum(agg + b_ref[...], 0.0)
    hb = _bn_in(h, g_ref[...], be_ref[...])
    o_ref[...] = dinv * jnp.dot(
        hb, W_ref[...], preferred_element_type=jnp.float32)

  return pl.pallas_call(
      body, out_shape=jax.ShapeDtypeStruct((N_NODES, HIDDEN), jnp.float32)
  )(p, u, degp, b, g, be, W)


def _tc_stage_c(p, u, degp, b3, batch2d, fg, fb, Wfc, bfc, hg, hbb, Wc, bc):
  def body(p_ref, u_ref, degp_ref, b3_ref, batch_ref, fg_ref, fb_ref, Wfc_ref,
           bfc_ref, hg_ref, hbb_ref, Wc_ref, bc_ref, o_ref):
    dinv = _dinv_in(degp_ref)
    h = jnp.maximum(
        dinv * (p_ref[0] + p_ref[1] + u_ref[...]) + b3_ref[...], 0.0)
    gid = lax.broadcasted_iota(jnp.int32, (N_GRAPHS, N_NODES), 0)
    msk = (gid == batch_ref[...]).astype(jnp.float32)
    pooled = jnp.dot(msk, h, preferred_element_type=jnp.float32)
    z = _bn_in(pooled, fg_ref[...], fb_ref[...])
    z = jnp.maximum(
        jnp.dot(z, Wfc_ref[...], preferred_element_type=jnp.float32)
        + bfc_ref[...], 0.0)
    z = _bn_in(z, hg_ref[...], hbb_ref[...])
    o_ref[...] = jnp.dot(
        z, Wc_ref[...], preferred_element_type=jnp.float32) + bc_ref[...]

  nc = bc.shape[0]
  return pl.pallas_call(
      body, out_shape=jax.ShapeDtypeStruct((N_GRAPHS, nc), jnp.float32)
  )(p, u, degp, b3, batch2d, fg, fb, Wfc, bfc, hg, hbb, Wc, bc)


def kernel(x, edge_index, batch, bn_feat_g, bn_feat_b, W_feat, b_feat, bn1_g,
           bn1_b, W1, b1, bn2_g, bn2_b, W2, b2, bn3_g, bn3_b, W3, b3, bnfc_g,
           bnfc_b, Wfc, bfc, bnh_g, bnh_b, Wc, bc):
  row3 = edge_index[0].astype(jnp.int32).reshape(_NW, _NB, _K)
  col3 = edge_index[1].astype(jnp.int32).reshape(_NW, _NB, _K)
  batch2d = batch.astype(jnp.int32).reshape(1, N_NODES)

  degp = _sc_deg(row3)
  u1 = _tc_stage_a(x, bn_feat_g, bn_feat_b, W_feat, b_feat, bn1_g, bn1_b, W1,
                   degp)
  p1 = _sc_agg(u1, row3, col3)
  u2 = _tc_stage_b(p1, u1, degp, b1, bn2_g, bn2_b, W2)
  p2 = _sc_agg(u2, row3, col3)
  u3 = _tc_stage_b(p2, u2, degp, b2, bn3_g, bn3_b, W3)
  p3 = _sc_agg(u3, row3, col3)
  return _tc_stage_c(p3, u3, degp, b3, batch2d, bnfc_g, bnfc_b, Wfc, bfc,
                     bnh_g, bnh_b, Wc, bc)


# speedup vs baseline: 13.1183x; 13.1183x over previous
import functools

import jax
import jax.numpy as jnp
from jax import lax
from jax.experimental import pallas as pl
from jax.experimental.pallas import tpu as pltpu
from jax.experimental.pallas import tpu_sc as plsc

N_NODES = 10000
N_EDGES = 320000
N_GRAPHS = 64
HIDDEN = 128
EPS = 1e-5

_NC = 2
_NS = 16
_NW = _NC * _NS
_K = 80
_NB = N_EDGES // (_NW * _K)
_RPT = 624
_EXTRA_BASE = _NS * _RPT
_EXTRA = N_NODES - _EXTRA_BASE

_mesh = plsc.VectorSubcoreMesh(
    core_axis_name="c", subcore_axis_name="s", num_cores=_NC, num_subcores=_NS
)


def _sc_agg(u, row3, col3):
  @functools.partial(
      pl.kernel,
      out_type=jax.ShapeDtypeStruct((_NC, N_NODES, HIDDEN), jnp.float32),
      mesh=_mesh,
      scratch_types=[
          pltpu.VMEM((_NB, _K), jnp.int32),
          pltpu.VMEM((_NB, _K), jnp.int32),
          pltpu.VMEM((_K, HIDDEN), jnp.float32),
          pltpu.VMEM_SHARED((N_NODES, HIDDEN), jnp.float32),
          pltpu.SemaphoreType.DMA,
      ],
  )
  def k(u_hbm, row_hbm, col_hbm, out_hbm, rowv, colv, gbuf, acc, sem):
    cid = lax.axis_index("c")
    sid = lax.axis_index("s")
    wid = cid * _NS + sid

    def fill_z(i, _):
      for c in range(HIDDEN // 16):
        gbuf[i, pl.ds(c * 16, 16)] = jnp.zeros((16,), jnp.float32)
      return 0

    lax.fori_loop(0, _K, fill_z, 0)
    base = sid * _RPT
    for j in range(_RPT // _K):
      pltpu.sync_copy(gbuf, acc.at[pl.ds(base + j * _K, _K)])
    pltpu.sync_copy(gbuf.at[pl.ds(0, _RPT % _K)],
                    acc.at[pl.ds(base + _RPT - _RPT % _K, _RPT % _K)])

    @pl.when(sid == _NS - 1)
    def _():
      pltpu.sync_copy(gbuf.at[pl.ds(0, _EXTRA)],
                      acc.at[pl.ds(_EXTRA_BASE, _EXTRA)])

    pltpu.sync_copy(row_hbm.at[wid], rowv)
    pltpu.sync_copy(col_hbm.at[wid], colv)
    plsc.subcore_barrier()

    def blk(i, _):
      pltpu.async_copy(u_hbm.at[colv.at[i]], gbuf, sem).wait()
      pltpu.sync_copy(gbuf, acc.at[rowv.at[i]], add=True)
      return 0

    lax.fori_loop(0, _NB, blk, 0)
    plsc.subcore_barrier()
    pltpu.sync_copy(
        acc.at[pl.ds(base, _RPT)],
        out_hbm.at[cid, pl.ds(base, _RPT)],
    )

    @pl.when(sid == _NS - 1)
    def _():
      pltpu.sync_copy(acc.at[pl.ds(_EXTRA_BASE, _EXTRA)],
                      out_hbm.at[cid, pl.ds(_EXTRA_BASE, _EXTRA)])

  return k(u, row3, col3)


def _bn_in(x, g, b):
  m = jnp.mean(x, axis=0)
  xc = x - m
  v = jnp.mean(xc * xc, axis=0)
  return g * xc * lax.rsqrt(v + EPS) + b


def _dinv_in(degp_ref):
  deg = degp_ref[0, :, 0:1] + degp_ref[1, :, 0:1] + 1.0
  return lax.rsqrt(deg)


def _tc_stage_a(x, bfg, bfb, Wf, bf, g1, b1b, W1, degp):
  def body(x_ref, bfg_ref, bfb_ref, Wf_ref, bf_ref, g1_ref, b1b_ref, W1_ref,
           degp_ref, u_ref):
    h = _bn_in(x_ref[...], bfg_ref[...], bfb_ref[...])
    h = jnp.maximum(
        jnp.dot(h, Wf_ref[...], preferred_element_type=jnp.float32)
        + bf_ref[...], 0.0)
    hb = _bn_in(h, g1_ref[...], b1b_ref[...])
    t = jnp.dot(hb, W1_ref[...], preferred_element_type=jnp.float32)
    u_ref[...] = _dinv_in(degp_ref) * t

  return pl.pallas_call(
      body, out_shape=jax.ShapeDtypeStruct((N_NODES, HIDDEN), jnp.float32)
  )(x, bfg, bfb, Wf, bf, g1, b1b, W1, degp)


def _tc_stage_b(p, u, degp, b, g, be, W):
  def body(p_ref, u_ref, degp_ref, b_ref, g_ref, be_ref, W_ref, o_ref):
    dinv = _dinv_in(degp_ref)
    agg = dinv * (p_ref[0] + p_ref[1] + u_ref[...])
    h = jnp.maximum(agg + b_ref[...], 0.0)
    hb = _bn_in(h, g_ref[...], be_ref[...])
    o_ref[...] = dinv * jnp.dot(
        hb, W_ref[...], preferred_element_type=jnp.float32)

  return pl.pallas_call(
      body, out_shape=jax.ShapeDtypeStruct((N_NODES, HIDDEN), jnp.float32)
  )(p, u, degp, b, g, be, W)


def _tc_stage_c(p, u, degp, b3, batch2d, fg, fb, Wfc, bfc, hg, hbb, Wc, bc):
  def body(p_ref, u_ref, degp_ref, b3_ref, batch_ref, fg_ref, fb_ref, Wfc_ref,
           bfc_ref, hg_ref, hbb_ref, Wc_ref, bc_ref, o_ref):
    dinv = _dinv_in(degp_ref)
    h = jnp.maximum(
        dinv * (p_ref[0] + p_ref[1] + u_ref[...]) + b3_ref[...], 0.0)
    gid = lax.broadcasted_iota(jnp.int32, (N_GRAPHS, N_NODES), 0)
    msk = (gid == batch_ref[...]).astype(jnp.float32)
    pooled = jnp.dot(msk, h, preferred_element_type=jnp.float32)
    z = _bn_in(pooled, fg_ref[...], fb_ref[...])
    z = jnp.maximum(
        jnp.dot(z, Wfc_ref[...], preferred_element_type=jnp.float32)
        + bfc_ref[...], 0.0)
    z = _bn_in(z, hg_ref[...], hbb_ref[...])
    o_ref[...] = jnp.dot(
        z, Wc_ref[...], preferred_element_type=jnp.float32) + bc_ref[...]

  nc = bc.shape[0]
  return pl.pallas_call(
      body, out_shape=jax.ShapeDtypeStruct((N_GRAPHS, nc), jnp.float32)
  )(p, u, degp, b3, batch2d, fg, fb, Wfc, bfc, hg, hbb, Wc, bc)


def kernel(x, edge_index, batch, bn_feat_g, bn_feat_b, W_feat, b_feat, bn1_g,
           bn1_b, W1, b1, bn2_g, bn2_b, W2, b2, bn3_g, bn3_b, W3, b3, bnfc_g,
           bnfc_b, Wfc, bfc, bnh_g, bnh_b, Wc, bc):
  row3 = edge_index[0].astype(jnp.int32).reshape(_NW, _NB, _K)
  col3 = edge_index[1].astype(jnp.int32).reshape(_NW, _NB, _K)
  batch2d = batch.astype(jnp.int32).reshape(1, N_NODES)

  ones = jnp.ones((N_NODES, HIDDEN), jnp.float32)
  degp = _sc_agg(ones, row3, col3)
  u1 = _tc_stage_a(x, bn_feat_g, bn_feat_b, W_feat, b_feat, bn1_g, bn1_b, W1,
                   degp)
  p1 = _sc_agg(u1, row3, col3)
  u2 = _tc_stage_b(p1, u1, degp, b1, bn2_g, bn2_b, W2)
  p2 = _sc_agg(u2, row3, col3)
  u3 = _tc_stage_b(p2, u2, degp, b2, bn3_g, bn3_b, W3)
  p3 = _sc_agg(u3, row3, col3)
  return _tc_stage_c(p3, u3, degp, b3, batch2d, bnfc_g, bnfc_b, Wfc, bfc,
                     bnh_g, bnh_b, Wc, bc)


# speedup vs baseline: 23.6292x; 1.8012x over previous
import functools

import jax
import jax.numpy as jnp
from jax import lax
from jax.experimental import pallas as pl
from jax.experimental.pallas import tpu as pltpu
from jax.experimental.pallas import tpu_sc as plsc

N_NODES = 10000
N_EDGES = 320000
N_GRAPHS = 64
HIDDEN = 128
EPS = 1e-5

_NC = 2
_NS = 16
_NW = _NC * _NS
_K = 80
_NB = N_EDGES // (_NW * _K)
_RPT = 624
_EXTRA_BASE = _NS * _RPT
_EXTRA = N_NODES - _EXTRA_BASE

_mesh = plsc.VectorSubcoreMesh(
    core_axis_name="c", subcore_axis_name="s", num_cores=_NC, num_subcores=_NS
)


def _zero_acc(zsrc, acc, sid, width):
  def fill_z(i, _):
    for c in range(width // 16):
      zsrc[i, pl.ds(c * 16, 16)] = jnp.zeros((16,), jnp.float32)
    return 0

  lax.fori_loop(0, _K, fill_z, 0)
  base = sid * _RPT
  for j in range(_RPT // _K):
    pltpu.sync_copy(zsrc, acc.at[pl.ds(base + j * _K, _K)])
  pltpu.sync_copy(zsrc.at[pl.ds(0, _RPT % _K)],
                  acc.at[pl.ds(base + _RPT - _RPT % _K, _RPT % _K)])

  @pl.when(sid == _NS - 1)
  def _():
    pltpu.sync_copy(zsrc.at[pl.ds(0, _EXTRA)],
                    acc.at[pl.ds(_EXTRA_BASE, _EXTRA)])


def _drain_acc(acc, out_hbm, cid, sid):
  base = sid * _RPT
  pltpu.sync_copy(acc.at[pl.ds(base, _RPT)],
                  out_hbm.at[cid, pl.ds(base, _RPT)])

  @pl.when(sid == _NS - 1)
  def _():
    pltpu.sync_copy(acc.at[pl.ds(_EXTRA_BASE, _EXTRA)],
                    out_hbm.at[cid, pl.ds(_EXTRA_BASE, _EXTRA)])


def _sc_agg(u, row3, colf):
  epw = _NB * _K

  @functools.partial(
      pl.kernel,
      out_type=jax.ShapeDtypeStruct((_NC, N_NODES, HIDDEN), jnp.float32),
      mesh=_mesh,
      scratch_types=[
          pltpu.VMEM((_NB, _K), jnp.int32),
          pltpu.VMEM((epw,), jnp.int32),
          pltpu.VMEM((_K, HIDDEN), jnp.float32),
          pltpu.VMEM((_K, HIDDEN), jnp.float32),
          pltpu.VMEM_SHARED((N_NODES, HIDDEN), jnp.float32),
          pltpu.SemaphoreType.DMA,
          pltpu.SemaphoreType.DMA,
          pltpu.SemaphoreType.DMA,
          pltpu.SemaphoreType.DMA,
      ],
  )
  def k(u_hbm, row_hbm, col_hbm, out_hbm, rowv, colv, g0, g1, acc,
        sg0, sg1, ss0, ss1):
    cid = lax.axis_index("c")
    sid = lax.axis_index("s")
    wid = cid * _NS + sid

    _zero_acc(g0, acc, sid, HIDDEN)
    pltpu.sync_copy(row_hbm.at[wid], rowv)
    pltpu.sync_copy(col_hbm.at[pl.ds(pl.multiple_of(wid * epw, 8), epw)],
                    colv)
    plsc.subcore_barrier()

    def gather(i, buf, sem):
      off = pl.multiple_of(i * _K, 8)
      pltpu.async_copy(u_hbm.at[colv.at[pl.ds(off, _K)]], buf, sem)

    def wait_gather(buf, sem):
      pltpu.make_async_copy(u_hbm.at[colv.at[pl.ds(0, _K)]], buf, sem).wait()

    def scatter(i, buf, sem):
      pltpu.async_copy(buf, acc.at[rowv.at[i]], sem, add=True)

    def wait_scatter(buf, sem):
      pltpu.make_async_copy(buf, acc.at[rowv.at[0]], sem).wait()

    gather(0, g0, sg0)
    gather(1, g1, sg1)
    wait_gather(g0, sg0)
    scatter(0, g0, ss0)

    def pair(j, _):
      i1 = 2 * j + 1
      wait_scatter(g0, ss0)
      gather(i1 + 1, g0, sg0)
      wait_gather(g1, sg1)
      scatter(i1, g1, ss1)
      i2 = i1 + 1
      wait_scatter(g1, ss1)
      gather(i2 + 1, g1, sg1)
      wait_gather(g0, sg0)
      scatter(i2, g0, ss0)
      return 0

    lax.fori_loop(0, (_NB - 3) // 2, pair, 0)

    wait_scatter(g0, ss0)
    gather(_NB - 1, g0, sg0)
    wait_gather(g1, sg1)
    scatter(_NB - 2, g1, ss1)
    wait_scatter(g1, ss1)
    wait_gather(g0, sg0)
    scatter(_NB - 1, g0, ss0)
    wait_scatter(g0, ss0)

    plsc.subcore_barrier()
    _drain_acc(acc, out_hbm, cid, sid)

  return k(u, row3, colf)


def _sc_deg(row3):
  @functools.partial(
      pl.kernel,
      out_type=jax.ShapeDtypeStruct((_NC, N_NODES, HIDDEN), jnp.float32),
      mesh=_mesh,
      scratch_types=[
          pltpu.VMEM((_NB, _K), jnp.int32),
          pltpu.VMEM((_K, HIDDEN), jnp.float32),
          pltpu.VMEM((_K, HIDDEN), jnp.float32),
          pltpu.VMEM_SHARED((N_NODES, HIDDEN), jnp.float32),
          pltpu.SemaphoreType.DMA,
      ],
  )
  def k(row_hbm, out_hbm, rowv, ones_v, zbuf, acc, ss):
    cid = lax.axis_index("c")
    sid = lax.axis_index("s")
    wid = cid * _NS + sid

    def fill_ones(i, _):
      for c in range(HIDDEN // 16):
        ones_v[i, pl.ds(c * 16, 16)] = jnp.full((16,), 1.0, jnp.float32)
      return 0

    lax.fori_loop(0, _K, fill_ones, 0)
    _zero_acc(zbuf, acc, sid, HIDDEN)
    pltpu.sync_copy(row_hbm.at[wid], rowv)
    plsc.subcore_barrier()

    def blk(i, _):
      pltpu.async_copy(ones_v, acc.at[rowv.at[i]], ss, add=True)
      return 0

    lax.fori_loop(0, _NB, blk, 0)

    def drain(i, _):
      pltpu.make_async_copy(ones_v, acc.at[rowv.at[0]], ss).wait()
      return 0

    lax.fori_loop(0, _NB, drain, 0)
    plsc.subcore_barrier()
    _drain_acc(acc, out_hbm, cid, sid)

  return k(row3)


def _bn_in(x, g, b):
  m = jnp.mean(x, axis=0)
  xc = x - m
  v = jnp.mean(xc * xc, axis=0)
  return g * xc * lax.rsqrt(v + EPS) + b


def _dinv_in(degp_ref):
  deg = degp_ref[0, :, 0:1] + degp_ref[1, :, 0:1] + 1.0
  return lax.rsqrt(deg)


def _tc_stage_a(x, bfg, bfb, Wf, bf, g1, b1b, W1, degp):
  def body(x_ref, bfg_ref, bfb_ref, Wf_ref, bf_ref, g1_ref, b1b_ref, W1_ref,
           degp_ref, u_ref):
    h = _bn_in(x_ref[...], bfg_ref[...], bfb_ref[...])
    h = jnp.maximum(
        jnp.dot(h, Wf_ref[...], preferred_element_type=jnp.float32)
        + bf_ref[...], 0.0)
    hb = _bn_in(h, g1_ref[...], b1b_ref[...])
    t = jnp.dot(hb, W1_ref[...], preferred_element_type=jnp.float32)
    u_ref[...] = _dinv_in(degp_ref) * t

  return pl.pallas_call(
      body, out_shape=jax.ShapeDtypeStruct((N_NODES, HIDDEN), jnp.float32)
  )(x, bfg, bfb, Wf, bf, g1, b1b, W1, degp)


def _tc_stage_b(p, u, degp, b, g, be, W):
  def body(p_ref, u_ref, degp_ref, b_ref, g_ref, be_ref, W_ref, o_ref):
    dinv = _dinv_in(degp_ref)
    agg = dinv * (p_ref[0] + p_ref[1] + u_ref[...])
    h = jnp.maximum(agg + b_ref[...], 0.0)
    hb = _bn_in(h, g_ref[...], be_ref[...])
    o_ref[...] = dinv * jnp.dot(
        hb, W_ref[...], preferred_element_type=jnp.float32)

  return pl.pallas_call(
      body, out_shape=jax.ShapeDtypeStruct((N_NODES, HIDDEN), jnp.float32)
  )(p, u, degp, b, g, be, W)


def _tc_stage_c(p, u, degp, b3, batch2d, fg, fb, Wfc, bfc, hg, hbb, Wc, bc):
  def body(p_ref, u_ref, degp_ref, b3_ref, batch_ref, fg_ref, fb_ref, Wfc_ref,
           bfc_ref, hg_ref, hbb_ref, Wc_ref, bc_ref, o_ref):
    dinv = _dinv_in(degp_ref)
    h = jnp.maximum(
        dinv * (p_ref[0] + p_ref[1] + u_ref[...]) + b3_ref[...], 0.0)
    gid = lax.broadcasted_iota(jnp.int32, (N_GRAPHS, N_NODES), 0)
    msk = (gid == batch_ref[...]).astype(jnp.float32)
    pooled = jnp.dot(msk, h, preferred_element_type=jnp.float32)
    z = _bn_in(pooled, fg_ref[...], fb_ref[...])
    z = jnp.maximum(
        jnp.dot(z, Wfc_ref[...], preferred_element_type=jnp.float32)
        + bfc_ref[...], 0.0)
    z = _bn_in(z, hg_ref[...], hbb_ref[...])
    o_ref[...] = jnp.dot(
        z, Wc_ref[...], preferred_element_type=jnp.float32) + bc_ref[...]

  nc = bc.shape[0]
  return pl.pallas_call(
      body, out_shape=jax.ShapeDtypeStruct((N_GRAPHS, nc), jnp.float32)
  )(p, u, degp, b3, batch2d, fg, fb, Wfc, bfc, hg, hbb, Wc, bc)


def kernel(x, edge_index, batch, bn_feat_g, bn_feat_b, W_feat, b_feat, bn1_g,
           bn1_b, W1, b1, bn2_g, bn2_b, W2, b2, bn3_g, bn3_b, W3, b3, bnfc_g,
           bnfc_b, Wfc, bfc, bnh_g, bnh_b, Wc, bc):
  row3 = edge_index[0].astype(jnp.int32).reshape(_NW, _NB, _K)
  colf = edge_index[1].astype(jnp.int32).reshape(_NW * _NB * _K)
  batch2d = batch.astype(jnp.int32).reshape(1, N_NODES)

  degp = _sc_deg(row3)
  u1 = _tc_stage_a(x, bn_feat_g, bn_feat_b, W_feat, b_feat, bn1_g, bn1_b, W1,
                   degp)
  p1 = _sc_agg(u1, row3, colf)
  u2 = _tc_stage_b(p1, u1, degp, b1, bn2_g, bn2_b, W2)
  p2 = _sc_agg(u2, row3, colf)
  u3 = _tc_stage_b(p2, u2, degp, b2, bn3_g, bn3_b, W3)
  p3 = _sc_agg(u3, row3, colf)
  return _tc_stage_c(p3, u3, degp, b3, batch2d, bnfc_g, bnfc_b, Wfc, bfc,
                     bnh_g, bnh_b, Wc, bc)


# speedup vs baseline: 24.5815x; 1.0403x over previous
import functools

import jax
import jax.numpy as jnp
from jax import lax
from jax.experimental import pallas as pl
from jax.experimental.pallas import tpu as pltpu
from jax.experimental.pallas import tpu_sc as plsc

N_NODES = 10000
N_EDGES = 320000
N_GRAPHS = 64
HIDDEN = 128
EPS = 1e-5

_NC = 2
_NS = 16
_NW = _NC * _NS
_K = 80
_NB = N_EDGES // (_NW * _K)
_RPT = 624
_EXTRA_BASE = _NS * _RPT
_EXTRA = N_NODES - _EXTRA_BASE

_mesh = plsc.VectorSubcoreMesh(
    core_axis_name="c", subcore_axis_name="s", num_cores=_NC, num_subcores=_NS
)


def _zero_acc(zsrc, acc, sid, width):
  def fill_z(i, _):
    for c in range(width // 16):
      zsrc[i, pl.ds(c * 16, 16)] = jnp.zeros((16,), jnp.float32)
    return 0

  lax.fori_loop(0, _K, fill_z, 0)
  base = sid * _RPT
  for j in range(_RPT // _K):
    pltpu.sync_copy(zsrc, acc.at[pl.ds(base + j * _K, _K)])
  pltpu.sync_copy(zsrc.at[pl.ds(0, _RPT % _K)],
                  acc.at[pl.ds(base + _RPT - _RPT % _K, _RPT % _K)])

  @pl.when(sid == _NS - 1)
  def _():
    pltpu.sync_copy(zsrc.at[pl.ds(0, _EXTRA)],
                    acc.at[pl.ds(_EXTRA_BASE, _EXTRA)])


def _drain_acc(acc, out_hbm, cid, sid):
  base = sid * _RPT
  pltpu.sync_copy(acc.at[pl.ds(base, _RPT)],
                  out_hbm.at[cid, pl.ds(base, _RPT)])

  @pl.when(sid == _NS - 1)
  def _():
    pltpu.sync_copy(acc.at[pl.ds(_EXTRA_BASE, _EXTRA)],
                    out_hbm.at[cid, pl.ds(_EXTRA_BASE, _EXTRA)])


_GB = 4
_NG = _NB // _GB
_REM_OFF = _NG * _GB * _K


def _sc_agg(u, row4, rowr, colf):
  epw = _NB * _K

  @functools.partial(
      pl.kernel,
      out_type=jax.ShapeDtypeStruct((_NC, N_NODES, HIDDEN), jnp.float32),
      mesh=_mesh,
      scratch_types=[
          pltpu.VMEM((_GB * _K,), jnp.int32),
          pltpu.VMEM((_GB * _K,), jnp.int32),
          pltpu.VMEM((_GB, _K), jnp.int32),
          pltpu.VMEM((_GB, _K), jnp.int32),
          pltpu.VMEM((1, _K), jnp.int32),
          pltpu.VMEM((_K, HIDDEN), jnp.float32),
          pltpu.VMEM((_K, HIDDEN), jnp.float32),
          pltpu.VMEM((_K, HIDDEN), jnp.float32),
          pltpu.VMEM((_K, HIDDEN), jnp.float32),
          pltpu.VMEM_SHARED((N_NODES, HIDDEN), jnp.float32),
          pltpu.SemaphoreType.DMA, pltpu.SemaphoreType.DMA,
          pltpu.SemaphoreType.DMA, pltpu.SemaphoreType.DMA,
          pltpu.SemaphoreType.DMA, pltpu.SemaphoreType.DMA,
          pltpu.SemaphoreType.DMA, pltpu.SemaphoreType.DMA,
          pltpu.SemaphoreType.DMA, pltpu.SemaphoreType.DMA,
      ],
  )
  def k(u_hbm, row_hbm, rowr_hbm, col_hbm, out_hbm,
        cb0, cb1, rb0, rb1, rbr, g0, g1, g2, g3, acc,
        sg0, sg1, sg2, sg3, ss0, ss1, ss2, ss3, si0, si1):
    gb = (g0, g1, g2, g3)
    sg = (sg0, sg1, sg2, sg3)
    ss = (ss0, ss1, ss2, ss3)
    cbs, rbs, sis = (cb0, cb1), (rb0, rb1), (si0, si1)
    cid = lax.axis_index("c")
    sid = lax.axis_index("s")
    wid = cid * _NS + sid
    cbase = wid * epw

    _zero_acc(g0, acc, sid, HIDDEN)

    def load_idx(g, p):
      off = pl.multiple_of(cbase + g * (_GB * _K), 8)
      pltpu.async_copy(col_hbm.at[pl.ds(off, _GB * _K)], cbs[p], sis[p])
      pltpu.async_copy(row_hbm.at[wid, g], rbs[p], sis[p])

    def wait_idx(p):
      pltpu.make_async_copy(
          col_hbm.at[pl.ds(0, _GB * _K)], cbs[p], sis[p]).wait()
      pltpu.make_async_copy(row_hbm.at[wid, 0], rbs[p], sis[p]).wait()

    def gather(q, p):
      pltpu.async_copy(
          u_hbm.at[cbs[p].at[pl.ds(q * _K, _K)]], gb[q], sg[q])

    def wait_g(q):
      pltpu.make_async_copy(
          u_hbm.at[cbs[0].at[pl.ds(0, _K)]], gb[q], sg[q]).wait()

    def scatter(q, p):
      pltpu.async_copy(gb[q], acc.at[rbs[p].at[q]], ss[q], add=True)

    def wait_s(q):
      pltpu.make_async_copy(gb[q], acc.at[rbs[0].at[0]], ss[q]).wait()

    load_idx(0, 0)
    wait_idx(0)
    plsc.subcore_barrier()
    for q in range(_GB):
      gather(q, 0)
    load_idx(1, 1)

    def two_groups(j, _):
      for q in range(_GB):
        wait_g(q)
        scatter(q, 0)
      wait_idx(1)
      for q in range(_GB):
        wait_s(q)
        gather(q, 1)
      load_idx(2 * j + 2, 0)
      for q in range(_GB):
        wait_g(q)
        scatter(q, 1)
      wait_idx(0)
      for q in range(_GB):
        wait_s(q)
        gather(q, 0)

      @pl.when(j < _NG // 2 - 1)
      def _():
        load_idx(2 * j + 3, 1)

      return 0

    lax.fori_loop(0, _NG // 2, two_groups, 0)

    for q in range(_GB):
      wait_g(q)
      scatter(q, 0)
    wait_s(0)
    off = pl.multiple_of(cbase + _REM_OFF, 8)
    pltpu.sync_copy(col_hbm.at[pl.ds(off, _K)], cb1.at[pl.ds(0, _K)])
    pltpu.sync_copy(rowr_hbm.at[wid], rbr)
    pltpu.async_copy(u_hbm.at[cb1.at[pl.ds(0, _K)]], g0, sg0)
    wait_g(0)
    pltpu.async_copy(g0, acc.at[rbr.at[0]], ss0, add=True)
    for q in range(1, _GB):
      wait_s(q)
    wait_s(0)

    plsc.subcore_barrier()
    _drain_acc(acc, out_hbm, cid, sid)

  return k(u, row4, rowr, colf)


def _sc_deg(row3):
  @functools.partial(
      pl.kernel,
      out_type=jax.ShapeDtypeStruct((_NC, N_NODES, HIDDEN), jnp.float32),
      mesh=_mesh,
      scratch_types=[
          pltpu.VMEM((_NB, _K), jnp.int32),
          pltpu.VMEM((_K, HIDDEN), jnp.float32),
          pltpu.VMEM((_K, HIDDEN), jnp.float32),
          pltpu.VMEM_SHARED((N_NODES, HIDDEN), jnp.float32),
          pltpu.SemaphoreType.DMA,
      ],
  )
  def k(row_hbm, out_hbm, rowv, ones_v, zbuf, acc, ss):
    cid = lax.axis_index("c")
    sid = lax.axis_index("s")
    wid = cid * _NS + sid

    def fill_ones(i, _):
      for c in range(HIDDEN // 16):
        ones_v[i, pl.ds(c * 16, 16)] = jnp.full((16,), 1.0, jnp.float32)
      return 0

    lax.fori_loop(0, _K, fill_ones, 0)
    _zero_acc(zbuf, acc, sid, HIDDEN)
    pltpu.sync_copy(row_hbm.at[wid], rowv)
    plsc.subcore_barrier()

    def blk(i, _):
      pltpu.async_copy(ones_v, acc.at[rowv.at[i]], ss, add=True)
      return 0

    lax.fori_loop(0, _NB, blk, 0)

    def drain(i, _):
      pltpu.make_async_copy(ones_v, acc.at[rowv.at[0]], ss).wait()
      return 0

    lax.fori_loop(0, _NB, drain, 0)
    plsc.subcore_barrier()
    _drain_acc(acc, out_hbm, cid, sid)

  return k(row3)


def _bn_in(x, g, b):
  m = jnp.mean(x, axis=0)
  xc = x - m
  v = jnp.mean(xc * xc, axis=0)
  return g * xc * lax.rsqrt(v + EPS) + b


def _dinv_in(degp_ref):
  deg = degp_ref[0, :, 0:1] + degp_ref[1, :, 0:1] + 1.0
  return lax.rsqrt(deg)


def _tc_stage_a(x, bfg, bfb, Wf, bf, g1, b1b, W1, degp):
  def body(x_ref, bfg_ref, bfb_ref, Wf_ref, bf_ref, g1_ref, b1b_ref, W1_ref,
           degp_ref, u_ref):
    h = _bn_in(x_ref[...], bfg_ref[...], bfb_ref[...])
    h = jnp.maximum(
        jnp.dot(h, Wf_ref[...], preferred_element_type=jnp.float32)
        + bf_ref[...], 0.0)
    hb = _bn_in(h, g1_ref[...], b1b_ref[...])
    t = jnp.dot(hb, W1_ref[...], preferred_element_type=jnp.float32)
    u_ref[...] = _dinv_in(degp_ref) * t

  return pl.pallas_call(
      body, out_shape=jax.ShapeDtypeStruct((N_NODES, HIDDEN), jnp.float32)
  )(x, bfg, bfb, Wf, bf, g1, b1b, W1, degp)


def _tc_stage_b(p, u, degp, b, g, be, W):
  def body(p_ref, u_ref, degp_ref, b_ref, g_ref, be_ref, W_ref, o_ref):
    dinv = _dinv_in(degp_ref)
    agg = dinv * (p_ref[0] + p_ref[1] + u_ref[...])
    h = jnp.maximum(agg + b_ref[...], 0.0)
    hb = _bn_in(h, g_ref[...], be_ref[...])
    o_ref[...] = dinv * jnp.dot(
        hb, W_ref[...], preferred_element_type=jnp.float32)

  return pl.pallas_call(
      body, out_shape=jax.ShapeDtypeStruct((N_NODES, HIDDEN), jnp.float32)
  )(p, u, degp, b, g, be, W)


def _tc_stage_c(p, u, degp, b3, batch2d, fg, fb, Wfc, bfc, hg, hbb, Wc, bc):
  def body(p_ref, u_ref, degp_ref, b3_ref, batch_ref, fg_ref, fb_ref, Wfc_ref,
           bfc_ref, hg_ref, hbb_ref, Wc_ref, bc_ref, o_ref):
    dinv = _dinv_in(degp_ref)
    h = jnp.maximum(
        dinv * (p_ref[0] + p_ref[1] + u_ref[...]) + b3_ref[...], 0.0)
    gid = lax.broadcasted_iota(jnp.int32, (N_GRAPHS, N_NODES), 0)
    msk = (gid == batch_ref[...]).astype(jnp.float32)
    pooled = jnp.dot(msk, h, preferred_element_type=jnp.float32)
    z = _bn_in(pooled, fg_ref[...], fb_ref[...])
    z = jnp.maximum(
        jnp.dot(z, Wfc_ref[...], preferred_element_type=jnp.float32)
        + bfc_ref[...], 0.0)
    z = _bn_in(z, hg_ref[...], hbb_ref[...])
    o_ref[...] = jnp.dot(
        z, Wc_ref[...], preferred_element_type=jnp.float32) + bc_ref[...]

  nc = bc.shape[0]
  return pl.pallas_call(
      body, out_shape=jax.ShapeDtypeStruct((N_GRAPHS, nc), jnp.float32)
  )(p, u, degp, b3, batch2d, fg, fb, Wfc, bfc, hg, hbb, Wc, bc)


def kernel(x, edge_index, batch, bn_feat_g, bn_feat_b, W_feat, b_feat, bn1_g,
           bn1_b, W1, b1, bn2_g, bn2_b, W2, b2, bn3_g, bn3_b, W3, b3, bnfc_g,
           bnfc_b, Wfc, bfc, bnh_g, bnh_b, Wc, bc):
  epw = _NB * _K
  row2 = edge_index[0].astype(jnp.int32).reshape(_NW, epw)
  row3 = row2.reshape(_NW, _NB, _K)
  row4 = row2[:, :_REM_OFF].reshape(_NW, _NG, _GB, _K)
  rowr = row2[:, _REM_OFF:].reshape(_NW, 1, _K)
  colf = edge_index[1].astype(jnp.int32).reshape(_NW * epw)
  batch2d = batch.astype(jnp.int32).reshape(1, N_NODES)

  degp = _sc_deg(row3)
  u1 = _tc_stage_a(x, bn_feat_g, bn_feat_b, W_feat, b_feat, bn1_g, bn1_b, W1,
                   degp)
  p1 = _sc_agg(u1, row4, rowr, colf)
  u2 = _tc_stage_b(p1, u1, degp, b1, bn2_g, bn2_b, W2)
  p2 = _sc_agg(u2, row4, rowr, colf)
  u3 = _tc_stage_b(p2, u2, degp, b2, bn3_g, bn3_b, W3)
  p3 = _sc_agg(u3, row4, rowr, colf)
  return _tc_stage_c(p3, u3, degp, b3, batch2d, bnfc_g, bnfc_b, Wfc, bfc,
                     bnh_g, bnh_b, Wc, bc)


# speedup vs baseline: 26.7793x; 1.0894x over previous
import functools

import jax
import jax.numpy as jnp
from jax import lax
from jax.experimental import pallas as pl
from jax.experimental.pallas import tpu as pltpu
from jax.experimental.pallas import tpu_sc as plsc

N_NODES = 10000
N_EDGES = 320000
N_GRAPHS = 64
HIDDEN = 128
EPS = 1e-5

_NC = 2
_NS = 16
_NW = _NC * _NS
_K = 80
_NB = N_EDGES // (_NW * _K)
_RPT = 624
_EXTRA_BASE = _NS * _RPT
_EXTRA = N_NODES - _EXTRA_BASE

_mesh = plsc.VectorSubcoreMesh(
    core_axis_name="c", subcore_axis_name="s", num_cores=_NC, num_subcores=_NS
)


def _zero_acc(zsrc, acc, sid, width):
  def fill_z(i, _):
    for c in range(width // 16):
      zsrc[i, pl.ds(c * 16, 16)] = jnp.zeros((16,), jnp.float32)
    return 0

  lax.fori_loop(0, _K, fill_z, 0)
  base = sid * _RPT
  for j in range(_RPT // _K):
    pltpu.sync_copy(zsrc, acc.at[pl.ds(base + j * _K, _K)])
  pltpu.sync_copy(zsrc.at[pl.ds(0, _RPT % _K)],
                  acc.at[pl.ds(base + _RPT - _RPT % _K, _RPT % _K)])

  @pl.when(sid == _NS - 1)
  def _():
    pltpu.sync_copy(zsrc.at[pl.ds(0, _EXTRA)],
                    acc.at[pl.ds(_EXTRA_BASE, _EXTRA)])


def _drain_acc(acc, out_hbm, cid, sid):
  base = sid * _RPT
  pltpu.sync_copy(acc.at[pl.ds(base, _RPT)],
                  out_hbm.at[cid, pl.ds(base, _RPT)])

  @pl.when(sid == _NS - 1)
  def _():
    pltpu.sync_copy(acc.at[pl.ds(_EXTRA_BASE, _EXTRA)],
                    out_hbm.at[cid, pl.ds(_EXTRA_BASE, _EXTRA)])


_GB = 4
_NG = _NB // _GB
_REM_OFF = _NG * _GB * _K


def _sc_agg(u, row4, rowr, colf):
  epw = _NB * _K

  @functools.partial(
      pl.kernel,
      out_type=jax.ShapeDtypeStruct((_NC, N_NODES, HIDDEN), jnp.float32),
      mesh=_mesh,
      scratch_types=[
          pltpu.VMEM((_GB * _K,), jnp.int32),
          pltpu.VMEM((_GB * _K,), jnp.int32),
          pltpu.VMEM((_GB, _K), jnp.int32),
          pltpu.VMEM((_GB, _K), jnp.int32),
          pltpu.VMEM((1, _K), jnp.int32),
          pltpu.VMEM((_K, HIDDEN), jnp.float32),
          pltpu.VMEM((_K, HIDDEN), jnp.float32),
          pltpu.VMEM((_K, HIDDEN), jnp.float32),
          pltpu.VMEM((_K, HIDDEN), jnp.float32),
          pltpu.VMEM_SHARED((N_NODES, HIDDEN), jnp.float32),
          pltpu.SemaphoreType.DMA, pltpu.SemaphoreType.DMA,
          pltpu.SemaphoreType.DMA, pltpu.SemaphoreType.DMA,
          pltpu.SemaphoreType.DMA, pltpu.SemaphoreType.DMA,
          pltpu.SemaphoreType.DMA, pltpu.SemaphoreType.DMA,
          pltpu.SemaphoreType.DMA, pltpu.SemaphoreType.DMA,
      ],
  )
  def k(u_hbm, row_hbm, rowr_hbm, col_hbm, out_hbm,
        cb0, cb1, rb0, rb1, rbr, g0, g1, g2, g3, acc,
        sg0, sg1, sg2, sg3, ss0, ss1, ss2, ss3, si0, si1):
    gb = (g0, g1, g2, g3)
    sg = (sg0, sg1, sg2, sg3)
    ss = (ss0, ss1, ss2, ss3)
    cbs, rbs, sis = (cb0, cb1), (rb0, rb1), (si0, si1)
    cid = lax.axis_index("c")
    sid = lax.axis_index("s")
    wid = cid * _NS + sid
    cbase = wid * epw

    _zero_acc(g0, acc, sid, HIDDEN)

    def load_idx(g, p):
      off = pl.multiple_of(cbase + g * (_GB * _K), 8)
      pltpu.async_copy(col_hbm.at[pl.ds(off, _GB * _K)], cbs[p], sis[p])
      pltpu.async_copy(row_hbm.at[wid, g], rbs[p], sis[p])

    def wait_idx(p):
      pltpu.make_async_copy(
          col_hbm.at[pl.ds(0, _GB * _K)], cbs[p], sis[p]).wait()
      pltpu.make_async_copy(row_hbm.at[wid, 0], rbs[p], sis[p]).wait()

    def gather(q, p):
      pltpu.async_copy(
          u_hbm.at[cbs[p].at[pl.ds(q * _K, _K)]], gb[q], sg[q])

    def wait_g(q):
      pltpu.make_async_copy(
          u_hbm.at[cbs[0].at[pl.ds(0, _K)]], gb[q], sg[q]).wait()

    def scatter(q, p):
      pltpu.async_copy(gb[q], acc.at[rbs[p].at[q]], ss[q], add=True)

    def wait_s(q):
      pltpu.make_async_copy(gb[q], acc.at[rbs[0].at[0]], ss[q]).wait()

    load_idx(0, 0)
    wait_idx(0)
    plsc.subcore_barrier()
    for q in range(_GB):
      gather(q, 0)
    load_idx(1, 1)

    def two_groups(j, _):
      for q in range(_GB):
        wait_g(q)
        scatter(q, 0)
      wait_idx(1)
      for q in range(_GB):
        wait_s(q)
        gather(q, 1)
      load_idx(2 * j + 2, 0)
      for q in range(_GB):
        wait_g(q)
        scatter(q, 1)
      wait_idx(0)
      for q in range(_GB):
        wait_s(q)
        gather(q, 0)

      @pl.when(j < _NG // 2 - 1)
      def _():
        load_idx(2 * j + 3, 1)

      return 0

    lax.fori_loop(0, _NG // 2, two_groups, 0)

    for q in range(_GB):
      wait_g(q)
      scatter(q, 0)
    wait_s(0)
    off = pl.multiple_of(cbase + _REM_OFF, 8)
    pltpu.sync_copy(col_hbm.at[pl.ds(off, _K)], cb1.at[pl.ds(0, _K)])
    pltpu.sync_copy(rowr_hbm.at[wid], rbr)
    pltpu.async_copy(u_hbm.at[cb1.at[pl.ds(0, _K)]], g0, sg0)
    wait_g(0)
    pltpu.async_copy(g0, acc.at[rbr.at[0]], ss0, add=True)
    for q in range(1, _GB):
      wait_s(q)
    wait_s(0)

    plsc.subcore_barrier()
    _drain_acc(acc, out_hbm, cid, sid)

  return k(u, row4, rowr, colf)


_DW = 16
_NDPAD = _NS * 640


def _sc_deg(row3):
  @functools.partial(
      pl.kernel,
      out_type=jax.ShapeDtypeStruct((_NC, _NS, 80, 128), jnp.float32),
      mesh=_mesh,
      scratch_types=[
          pltpu.VMEM((_NB, _K), jnp.int32),
          pltpu.VMEM((_K, _DW), jnp.float32),
          pltpu.VMEM((_K, _DW), jnp.float32),
          pltpu.VMEM((80, 128), jnp.float32),
          pltpu.VMEM_SHARED((_NDPAD, _DW), jnp.float32),
          pltpu.SemaphoreType.DMA,
      ],
  )
  def k(row_hbm, out_hbm, rowv, ones_v, dbuf, pbuf, acc, ss):
    cid = lax.axis_index("c")
    sid = lax.axis_index("s")
    wid = cid * _NS + sid

    def fill(i, _):
      ones_v[i, :] = jnp.full((_DW,), 1.0, jnp.float32)
      dbuf[i, :] = jnp.zeros((_DW,), jnp.float32)
      return 0

    lax.fori_loop(0, _K, fill, 0)
    base = sid * 640
    for j in range(640 // _K):
      pltpu.sync_copy(dbuf, acc.at[pl.ds(base + j * _K, _K)])
    pltpu.sync_copy(row_hbm.at[wid], rowv)
    plsc.subcore_barrier()

    def blk(i, _):
      pltpu.async_copy(ones_v, acc.at[rowv.at[i]], ss, add=True)
      return 0

    lax.fori_loop(0, _NB, blk, 0)

    def drain(i, _):
      pltpu.make_async_copy(ones_v, acc.at[rowv.at[0]], ss).wait()
      return 0

    lax.fori_loop(0, _NB, drain, 0)
    plsc.subcore_barrier()

    for c in range(8):
      pltpu.sync_copy(acc.at[pl.ds(base + c * _K, _K)], dbuf)
      for r in range(10):
        for s in range(8):
          pbuf[c * 10 + r, pl.ds(s * _DW, _DW)] = dbuf[8 * r + s, :]
    pltpu.sync_copy(pbuf, out_hbm.at[cid, sid])

  return k(row3)


def _bn_in(x, g, b):
  m = jnp.mean(x, axis=0)
  xc = x - m
  v = jnp.mean(xc * xc, axis=0)
  return g * xc * lax.rsqrt(v + EPS) + b


def _dinv_in(degp_ref):
  deg = (degp_ref[0, 0:N_NODES, 0:1] + degp_ref[1, 0:N_NODES, 0:1]) + 1.0
  return lax.rsqrt(deg)


def _tc_stage_a(x, bfg, bfb, Wf, bf, g1, b1b, W1, degp):
  def body(x_ref, bfg_ref, bfb_ref, Wf_ref, bf_ref, g1_ref, b1b_ref, W1_ref,
           degp_ref, u_ref):
    h = _bn_in(x_ref[...], bfg_ref[...], bfb_ref[...])
    h = jnp.maximum(
        jnp.dot(h, Wf_ref[...], preferred_element_type=jnp.float32)
        + bf_ref[...], 0.0)
    hb = _bn_in(h, g1_ref[...], b1b_ref[...])
    t = jnp.dot(hb, W1_ref[...], preferred_element_type=jnp.float32)
    u_ref[...] = _dinv_in(degp_ref) * t

  return pl.pallas_call(
      body, out_shape=jax.ShapeDtypeStruct((N_NODES, HIDDEN), jnp.float32)
  )(x, bfg, bfb, Wf, bf, g1, b1b, W1, degp)


def _tc_stage_b(p, u, degp, b, g, be, W):
  def body(p_ref, u_ref, degp_ref, b_ref, g_ref, be_ref, W_ref, o_ref):
    dinv = _dinv_in(degp_ref)
    agg = dinv * (p_ref[0] + p_ref[1] + u_ref[...])
    h = jnp.maximum(agg + b_ref[...], 0.0)
    hb = _bn_in(h, g_ref[...], be_ref[...])
    o_ref[...] = dinv * jnp.dot(
        hb, W_ref[...], preferred_element_type=jnp.float32)

  return pl.pallas_call(
      body, out_shape=jax.ShapeDtypeStruct((N_NODES, HIDDEN), jnp.float32)
  )(p, u, degp, b, g, be, W)


def _tc_stage_c(p, u, degp, b3, batch2d, fg, fb, Wfc, bfc, hg, hbb, Wc, bc):
  def body(p_ref, u_ref, degp_ref, b3_ref, batch_ref, fg_ref, fb_ref, Wfc_ref,
           bfc_ref, hg_ref, hbb_ref, Wc_ref, bc_ref, o_ref):
    dinv = _dinv_in(degp_ref)
    h = jnp.maximum(
        dinv * (p_ref[0] + p_ref[1] + u_ref[...]) + b3_ref[...], 0.0)
    gid = lax.broadcasted_iota(jnp.int32, (N_GRAPHS, N_NODES), 0)
    msk = (gid == batch_ref[...]).astype(jnp.float32)
    pooled = jnp.dot(msk, h, preferred_element_type=jnp.float32)
    z = _bn_in(pooled, fg_ref[...], fb_ref[...])
    z = jnp.maximum(
        jnp.dot(z, Wfc_ref[...], preferred_element_type=jnp.float32)
        + bfc_ref[...], 0.0)
    z = _bn_in(z, hg_ref[...], hbb_ref[...])
    o_ref[...] = jnp.dot(
        z, Wc_ref[...], preferred_element_type=jnp.float32) + bc_ref[...]

  nc = bc.shape[0]
  return pl.pallas_call(
      body, out_shape=jax.ShapeDtypeStruct((N_GRAPHS, nc), jnp.float32)
  )(p, u, degp, b3, batch2d, fg, fb, Wfc, bfc, hg, hbb, Wc, bc)


def kernel(x, edge_index, batch, bn_feat_g, bn_feat_b, W_feat, b_feat, bn1_g,
           bn1_b, W1, b1, bn2_g, bn2_b, W2, b2, bn3_g, bn3_b, W3, b3, bnfc_g,
           bnfc_b, Wfc, bfc, bnh_g, bnh_b, Wc, bc):
  epw = _NB * _K
  row2 = edge_index[0].astype(jnp.int32).reshape(_NW, epw)
  row3 = row2.reshape(_NW, _NB, _K)
  row4 = row2[:, :_REM_OFF].reshape(_NW, _NG, _GB, _K)
  rowr = row2[:, _REM_OFF:].reshape(_NW, 1, _K)
  colf = edge_index[1].astype(jnp.int32).reshape(_NW * epw)
  batch2d = batch.astype(jnp.int32).reshape(1, N_NODES)

  degp = _sc_deg(row3).reshape(_NC, _NDPAD, _DW)
  u1 = _tc_stage_a(x, bn_feat_g, bn_feat_b, W_feat, b_feat, bn1_g, bn1_b, W1,
                   degp)
  p1 = _sc_agg(u1, row4, rowr, colf)
  u2 = _tc_stage_b(p1, u1, degp, b1, bn2_g, bn2_b, W2)
  p2 = _sc_agg(u2, row4, rowr, colf)
  u3 = _tc_stage_b(p2, u2, degp, b2, bn3_g, bn3_b, W3)
  p3 = _sc_agg(u3, row4, rowr, colf)
  return _tc_stage_c(p3, u3, degp, b3, batch2d, bnfc_g, bnfc_b, Wfc, bfc,
                     bnh_g, bnh_b, Wc, bc)
